# baseline (speedup 1.0000x reference)
import functools

import jax
import jax.numpy as jnp
from jax import lax
from jax.experimental import pallas as pl
from jax.experimental.pallas import tpu as pltpu
from jax.experimental.pallas import tpu_sc as plsc

NC, NS, LANES = 2, 16, 16
NW = NC * NS
B = 1024
T = 1000
TPAD = 1024
CHUNK = 128
NCHUNK = TPAD // CHUNK
D = 32
BPW = B // NW
NGRP = TPAD // LANES
NGRP_FULL = T // LANES
BIG = 2**30


def _fire(table_hbm, idx_all, rows, sem, i):
    return [
        pltpu.async_copy(table_hbm.at[idx_all.at[i, j]],
                         rows.at[pl.ds(j * CHUNK, CHUNK)], sem)
        for j in range(NCHUNK)
    ]


def _drain(table_hbm, idx_all, rows, sem, i):
    for j in range(NCHUNK):
        pltpu.make_async_copy(table_hbm.at[idx_all.at[i, j]],
                              rows.at[pl.ds(j * CHUNK, CHUNK)], sem).wait()


def _batch_compute(rows, colbuf, normbuf, feat_v, i, iota):
    zeros = jnp.zeros((LANES,), jnp.float32)
    half = D // 2

    def body1(g, carry):
        colacc = list(carry)
        base = g * LANES
        row_idx = base + iota
        nacc = zeros
        for d in range(half):
            v = plsc.load_gather(rows, [row_idx, jnp.full((LANES,), d, jnp.int32)])
            colacc[d] = colacc[d] + v
            nacc = nacc + v * v
        normbuf[g, pl.ds(0, LANES)] = nacc
        return tuple(colacc)

    colacc_lo = list(lax.fori_loop(0, NGRP, body1, tuple([zeros] * half)))

    def group2(base, colacc, m16, bi16, g, do_max, mask8):
        row_idx = base + iota
        nacc = normbuf[g, pl.ds(0, LANES)]
        for d in range(half):
            v = plsc.load_gather(rows,
                                 [row_idx, jnp.full((LANES,), half + d, jnp.int32)])
            colacc[d] = colacc[d] + v
            nacc = nacc + v * v
        if do_max:
            if mask8:
                nacc = jnp.where(iota < 8, nacc, jnp.float32(-1.0))
            pred = nacc > m16
            m16 = jnp.where(pred, nacc, m16)
            bi16 = jnp.where(pred, row_idx, bi16)
        return colacc, m16, bi16

    def body2(g, carry):
        m16, bi16 = carry[0], carry[1]
        colacc = list(carry[2:])
        colacc, m16, bi16 = group2(g * LANES, colacc, m16, bi16, g, True, False)
        return (m16, bi16, *colacc)

    init = (jnp.full((LANES,), -1.0, jnp.float32), jnp.zeros((LANES,), jnp.int32),
            *([zeros] * half))
    carry = lax.fori_loop(0, NGRP_FULL, body2, init)
    m16, bi16 = carry[0], carry[1]
    colacc_hi = list(carry[2:])
    colacc_hi, m16, bi16 = group2(jnp.int32(NGRP_FULL * LANES), colacc_hi, m16,
                                  bi16, jnp.int32(NGRP_FULL), True, True)
    colacc_hi, m16, bi16 = group2(jnp.int32((NGRP_FULL + 1) * LANES), colacc_hi,
                                  m16, bi16, jnp.int32(NGRP_FULL + 1), False,
                                  False)

    for d in range(half):
        colbuf[d, pl.ds(0, LANES)] = colacc_lo[d]
        colbuf[half + d, pl.ds(0, LANES)] = colacc_hi[d]
    pad0 = rows[TPAD - LANES, pl.ds(0, LANES)]
    pad1 = rows[TPAD - LANES, pl.ds(LANES, LANES)]
    t0 = jnp.float32(-24.0) * pad0
    t1 = jnp.float32(-24.0) * pad1
    for l in range(LANES):
        lane = jnp.full((LANES,), l, jnp.int32)
        t0 = t0 + plsc.load_gather(colbuf, [iota, lane])
        t1 = t1 + plsc.load_gather(colbuf, [LANES + iota, lane])

    mmax = jnp.max(m16)
    cand = jnp.where(m16 == mmax, bi16, jnp.int32(BIG))
    bi = jnp.min(cand)
    best0 = rows[bi, pl.ds(0, LANES)]
    best1 = rows[bi, pl.ds(LANES, LANES)]

    inv = jnp.float32(1.0 / T)
    off = i * (2 * D)
    feat_v[pl.ds(off, LANES)] = t0 * inv
    feat_v[pl.ds(off + LANES, LANES)] = t1 * inv
    feat_v[pl.ds(off + 2 * LANES, LANES)] = best0
    feat_v[pl.ds(off + 3 * LANES, LANES)] = best1


def _sc_body(idx_hbm, table_hbm, w_hbm, b_hbm, out_hbm,
             idx_all, rows_a, rows_b, colbuf, normbuf, feat_v, w_v, bias_v,
             out_v, sem_a, sem_b):
    wid = lax.axis_index("s") * NC + lax.axis_index("c")
    base = wid * BPW
    iota = lax.iota(jnp.int32, LANES)

    pltpu.sync_copy(w_hbm, w_v)
    pltpu.sync_copy(b_hbm, bias_v)
    pltpu.sync_copy(idx_hbm.at[pl.ds(base, BPW)], idx_all)

    _fire(table_hbm, idx_all, rows_a, sem_a, 0)

    def pair_body(k, carry):
        i0 = 2 * k
        _drain(table_hbm, idx_all, rows_a, sem_a, i0)
        _fire(table_hbm, idx_all, rows_b, sem_b, i0 + 1)
        _batch_compute(rows_a, colbuf, normbuf, feat_v, i0, iota)
        _drain(table_hbm, idx_all, rows_b, sem_b, i0 + 1)

        @pl.when(k < BPW // 2 - 1)
        def _():
            _fire(table_hbm, idx_all, rows_a, sem_a, i0 + 2)

        _batch_compute(rows_b, colbuf, normbuf, feat_v, i0 + 1, iota)
        return carry

    lax.fori_loop(0, BPW // 2, pair_body, 0)

    bvec = bias_v[pl.ds(0, LANES)]
    b0 = bvec[0]
    b1 = bvec[1]
    w0vecs = [w_v[pl.ds(k * LANES, LANES)] for k in range(2 * D // LANES)]
    w1vecs = [w_v[pl.ds(2 * D + k * LANES, LANES)] for k in range(2 * D // LANES)]
    iota_feat = iota * (2 * D)
    for g in range(BPW // LANES):
        acc0 = jnp.broadcast_to(b0, (LANES,))
        acc1 = jnp.broadcast_to(b1, (LANES,))
        gbase = g * LANES * (2 * D)
        for d in range(2 * D):
            v = plsc.load_gather(feat_v, [iota_feat + (gbase + d)])
            acc0 = acc0 + v * w0vecs[d // LANES][d % LANES]
            acc1 = acc1 + v * w1vecs[d // LANES][d % LANES]
        p0 = 1.0 / (1.0 + jnp.exp(-acc0))
        p1 = 1.0 / (1.0 + jnp.exp(-acc1))
        row_idx = g * LANES + iota
        plsc.store_scatter(out_v, [row_idx, jnp.zeros((LANES,), jnp.int32)], p0)
        plsc.store_scatter(out_v, [row_idx, jnp.ones((LANES,), jnp.int32)], p1)

    pltpu.sync_copy(out_v, out_hbm.at[pl.ds(base, BPW)])


@jax.jit
def _logreg_sc(idx3, table, wflat, bpad):
    mesh = plsc.VectorSubcoreMesh(core_axis_name="c", subcore_axis_name="s",
                                  num_cores=NC, num_subcores=NS)
    fn = pl.kernel(
        _sc_body,
        out_type=jax.ShapeDtypeStruct((B, 2), jnp.float32),
        mesh=mesh,
        compiler_params=pltpu.CompilerParams(needs_layout_passes=False,
                                             use_tc_tiling_on_sc=False),
        scratch_types=[
            pltpu.VMEM((BPW, NCHUNK, CHUNK), jnp.int32),
            pltpu.VMEM((TPAD, D), jnp.float32),
            pltpu.VMEM((TPAD, D), jnp.float32),
            pltpu.VMEM((D, LANES), jnp.float32),
            pltpu.VMEM((NGRP, LANES), jnp.float32),
            pltpu.VMEM((BPW * 2 * D,), jnp.float32),
            pltpu.VMEM((2 * 2 * D,), jnp.float32),
            pltpu.VMEM((LANES,), jnp.float32),
            pltpu.VMEM((BPW, 2), jnp.float32),
            pltpu.SemaphoreType.DMA,
            pltpu.SemaphoreType.DMA,
        ],
    )
    return fn(idx3, table, wflat, bpad)


def kernel(indices, embedding_matrix, dense_W, dense_b):
    idx = indices.reshape(B, T).astype(jnp.int32)
    idxp = jnp.pad(idx, ((0, 0), (0, TPAD - T)))
    idx3 = idxp.reshape(B, NCHUNK, CHUNK)
    wflat = dense_W.astype(jnp.float32).T.reshape(2 * 2 * D)
    bpad = jnp.pad(dense_b.astype(jnp.float32), (0, LANES - 2))
    return _logreg_sc(idx3, embedding_matrix, wflat, bpad)


# speedup vs baseline: 1.0199x; 1.0199x over previous
import functools

import jax
import jax.numpy as jnp
from jax import lax
from jax.experimental import pallas as pl
from jax.experimental.pallas import tpu as pltpu
from jax.experimental.pallas import tpu_sc as plsc

NC, NS, LANES = 2, 16, 16
NW = NC * NS
B = 1024
T = 1000
TPAD = 1024
CHUNK = 128
NCHUNK = TPAD // CHUNK
D = 32
BPW = B // NW
NGRP = TPAD // LANES
NGRP_FULL = T // LANES
BIG = 2**30


def _fire(table_hbm, idx_all, rows, sem, i):
    return [
        pltpu.async_copy(table_hbm.at[idx_all.at[i, j]],
                         rows.at[pl.ds(j * CHUNK, CHUNK)], sem)
        for j in range(NCHUNK)
    ]


def _drain(table_hbm, idx_all, rows, sem, i):
    for j in range(NCHUNK):
        pltpu.make_async_copy(table_hbm.at[idx_all.at[i, j]],
                              rows.at[pl.ds(j * CHUNK, CHUNK)], sem).wait()


def _batch_compute(rows, normbuf, feat_v, i, iota):
    zeros = jnp.zeros((LANES,), jnp.float32)
    unroll = 8

    def body1(it, carry):
        s = list(carry)
        for u in range(unroll):
            r = it * unroll + u
            a = rows[r, pl.ds(0, LANES)]
            b = rows[r, pl.ds(LANES, LANES)]
            s[2 * (u % 2)] = s[2 * (u % 2)] + a
            s[2 * (u % 2) + 1] = s[2 * (u % 2) + 1] + b
            c = a * a + b * b
            normbuf[r, pl.ds(0, LANES)] = jnp.cumsum(c)
        return tuple(s)

    s = lax.fori_loop(0, T // unroll, body1, (zeros,) * 4)
    t0 = s[0] + s[2]
    t1 = s[1] + s[3]

    lane15 = jnp.full((LANES,), LANES - 1, jnp.int32)

    def body2(g, carry):
        m16, bi16 = carry
        row_idx = g * LANES + iota
        nv = plsc.load_gather(normbuf, [row_idx, lane15])
        pred = nv > m16
        m16 = jnp.where(pred, nv, m16)
        bi16 = jnp.where(pred, row_idx, bi16)
        return m16, bi16

    m16, bi16 = lax.fori_loop(0, NGRP_FULL, body2,
                              (jnp.full((LANES,), -1.0, jnp.float32),
                               jnp.zeros((LANES,), jnp.int32)))
    row_idx = jnp.int32(NGRP_FULL * LANES) + iota
    nv = plsc.load_gather(normbuf, [row_idx, lane15])
    nv = jnp.where(iota < 8, nv, jnp.float32(-1.0))
    pred = nv > m16
    m16 = jnp.where(pred, nv, m16)
    bi16 = jnp.where(pred, row_idx, bi16)

    mmax = jnp.max(m16)
    cand = jnp.where(m16 == mmax, bi16, jnp.int32(BIG))
    bi = jnp.min(cand)
    best0 = rows[bi, pl.ds(0, LANES)]
    best1 = rows[bi, pl.ds(LANES, LANES)]

    inv = jnp.float32(1.0 / T)
    off = i * (2 * D)
    feat_v[pl.ds(off, LANES)] = t0 * inv
    feat_v[pl.ds(off + LANES, LANES)] = t1 * inv
    feat_v[pl.ds(off + 2 * LANES, LANES)] = best0
    feat_v[pl.ds(off + 3 * LANES, LANES)] = best1


def _sc_body(idx_hbm, table_hbm, w_hbm, b_hbm, out_hbm,
             idx_all, rows_a, rows_b, normbuf, feat_v, w_v, bias_v,
             out_v, sem_a, sem_b):
    wid = lax.axis_index("s") * NC + lax.axis_index("c")
    base = wid * BPW
    iota = lax.iota(jnp.int32, LANES)

    pltpu.sync_copy(w_hbm, w_v)
    pltpu.sync_copy(b_hbm, bias_v)
    pltpu.sync_copy(idx_hbm.at[pl.ds(base, BPW)], idx_all)

    _fire(table_hbm, idx_all, rows_a, sem_a, 0)

    def pair_body(k, carry):
        i0 = 2 * k
        _drain(table_hbm, idx_all, rows_a, sem_a, i0)
        _fire(table_hbm, idx_all, rows_b, sem_b, i0 + 1)
        _batch_compute(rows_a, normbuf, feat_v, i0, iota)
        _drain(table_hbm, idx_all, rows_b, sem_b, i0 + 1)

        @pl.when(k < BPW // 2 - 1)
        def _():
            _fire(table_hbm, idx_all, rows_a, sem_a, i0 + 2)

        _batch_compute(rows_b, normbuf, feat_v, i0 + 1, iota)
        return carry

    lax.fori_loop(0, BPW // 2, pair_body, 0)

    bvec = bias_v[pl.ds(0, LANES)]
    b0 = bvec[0]
    b1 = bvec[1]
    w0vecs = [w_v[pl.ds(k * LANES, LANES)] for k in range(2 * D // LANES)]
    w1vecs = [w_v[pl.ds(2 * D + k * LANES, LANES)] for k in range(2 * D // LANES)]
    iota_feat = iota * (2 * D)
    for g in range(BPW // LANES):
        acc0 = jnp.broadcast_to(b0, (LANES,))
        acc1 = jnp.broadcast_to(b1, (LANES,))
        gbase = g * LANES * (2 * D)
        for d in range(2 * D):
            v = plsc.load_gather(feat_v, [iota_feat + (gbase + d)])
            acc0 = acc0 + v * w0vecs[d // LANES][d % LANES]
            acc1 = acc1 + v * w1vecs[d // LANES][d % LANES]
        p0 = 1.0 / (1.0 + jnp.exp(-acc0))
        p1 = 1.0 / (1.0 + jnp.exp(-acc1))
        row_idx = g * LANES + iota
        plsc.store_scatter(out_v, [row_idx, jnp.zeros((LANES,), jnp.int32)], p0)
        plsc.store_scatter(out_v, [row_idx, jnp.ones((LANES,), jnp.int32)], p1)

    pltpu.sync_copy(out_v, out_hbm.at[pl.ds(base, BPW)])


@jax.jit
def _logreg_sc(idx3, table, wflat, bpad):
    mesh = plsc.VectorSubcoreMesh(core_axis_name="c", subcore_axis_name="s",
                                  num_cores=NC, num_subcores=NS)
    fn = pl.kernel(
        _sc_body,
        out_type=jax.ShapeDtypeStruct((B, 2), jnp.float32),
        mesh=mesh,
        compiler_params=pltpu.CompilerParams(needs_layout_passes=False,
                                             use_tc_tiling_on_sc=False),
        scratch_types=[
            pltpu.VMEM((BPW, NCHUNK, CHUNK), jnp.int32),
            pltpu.VMEM((TPAD, D), jnp.float32),
            pltpu.VMEM((TPAD, D), jnp.float32),
            pltpu.VMEM((TPAD, LANES), jnp.float32),
            pltpu.VMEM((BPW * 2 * D,), jnp.float32),
            pltpu.VMEM((2 * 2 * D,), jnp.float32),
            pltpu.VMEM((LANES,), jnp.float32),
            pltpu.VMEM((BPW, 2), jnp.float32),
            pltpu.SemaphoreType.DMA,
            pltpu.SemaphoreType.DMA,
        ],
    )
    return fn(idx3, table, wflat, bpad)


def kernel(indices, embedding_matrix, dense_W, dense_b):
    idx = indices.reshape(B, T).astype(jnp.int32)
    idxp = jnp.pad(idx, ((0, 0), (0, TPAD - T)))
    idx3 = idxp.reshape(B, NCHUNK, CHUNK)
    wflat = dense_W.astype(jnp.float32).T.reshape(2 * 2 * D)
    bpad = jnp.pad(dense_b.astype(jnp.float32), (0, LANES - 2))
    return _logreg_sc(idx3, embedding_matrix, wflat, bpad)
